# baseline (speedup 1.0000x reference)
import functools

import jax
import jax.numpy as jnp
from jax import lax
from jax.experimental import pallas as pl
from jax.experimental.pallas import tpu as pltpu
from jax.experimental.pallas import tpu_sc as plsc

N = 10000
E = 160000
D = 256
L = 3
G = 64
NA, VA, NB, VB = 9, 119, 3, 6

H = 128
NT = 16
NPAD = 10240
RPT = NPAD // NT
EPT = E // NT
CH = 80
NCH = EPT // CH
AC = 8
CHA = RPT // AC

_mesh = plsc.VectorSubcoreMesh(core_axis_name="c", subcore_axis_name="s")


@functools.partial(
    pl.kernel,
    out_type=jax.ShapeDtypeStruct((2, NPAD, H), jnp.float32),
    mesh=_mesh,
    scratch_types=[
        pltpu.VMEM((NA, AC, CHA), jnp.int32),
        pltpu.VMEM((RPT, H), jnp.float32),
        pltpu.SemaphoreType.DMA,
        pltpu.SemaphoreType.DMA,
    ],
)
def _atom_sc(xg_hbm, tab_hbm, out_hbm, idx_v, buf, sem_t, sem_a):
    c = lax.axis_index("c")
    s = lax.axis_index("s")
    pltpu.sync_copy(xg_hbm.at[s], idx_v)

    def _f0(k):
        pltpu.async_copy(
            tab_hbm.at[c, 0].at[idx_v.at[0, k]],
            buf.at[pl.ds(k * CHA, CHA)], sem_t)

    _f0(0)

    def chunk(k, carry):
        pltpu.make_async_copy(
            tab_hbm.at[c, 0].at[idx_v.at[0, k]],
            buf.at[pl.ds(k * CHA, CHA)], sem_t).wait()

        @pl.when(k + 1 < AC)
        def _():
            _f0(k + 1)

        for f in range(1, NA):
            pltpu.async_copy(
                tab_hbm.at[c, f].at[idx_v.at[f, k]],
                buf.at[pl.ds(k * CHA, CHA)], sem_a, add=True)
        return carry

    lax.fori_loop(0, AC, chunk, 0)

    def drain(k, carry):
        pltpu.make_async_copy(
            tab_hbm.at[c, 1].at[idx_v.at[1, 0]],
            buf.at[pl.ds(0, CHA)], sem_a).wait()
        return carry

    lax.fori_loop(0, AC * (NA - 1), drain, 0)
    pltpu.sync_copy(buf, out_hbm.at[c, pl.ds(s * RPT, RPT)])


@functools.partial(
    pl.kernel,
    out_type=jax.ShapeDtypeStruct((2, NPAD, H), jnp.float32),
    mesh=_mesh,
    scratch_types=[
        pltpu.VMEM((3, 3, CH), jnp.int32),
        pltpu.VMEM((2, CH, H), jnp.float32),
        pltpu.VMEM_SHARED((NPAD, H), jnp.float32),
        pltpu.SemaphoreType.DMA,
        pltpu.SemaphoreType.DMA,
    ],
)
def _edge_sc(hin_hbm, eg_hbm, tab_hbm, zrows_hbm, out_hbm,
             idx3, buf, aggr, sem_t, sem_a):
    c = lax.axis_index("c")
    s = lax.axis_index("s")
    pltpu.sync_copy(zrows_hbm, aggr.at[pl.ds(s * RPT, RPT)])
    plsc.subcore_barrier()

    def _idx(k):
        pltpu.sync_copy(eg_hbm.at[s, k], idx3.at[k % 3])

    def _tab(k):
        pltpu.async_copy(tab_hbm.at[c].at[idx3.at[k % 3, 2]],
                         buf.at[k % 2], sem_t)

    def _wait_tab(k):
        pltpu.make_async_copy(tab_hbm.at[c].at[idx3.at[k % 3, 2]],
                              buf.at[k % 2], sem_t).wait()

    def _add(k):
        pltpu.async_copy(hin_hbm.at[c].at[idx3.at[k % 3, 0]],
                         buf.at[k % 2], sem_a, add=True)

    def _wait_add(k):
        pltpu.make_async_copy(hin_hbm.at[c].at[idx3.at[k % 3, 0]],
                              buf.at[k % 2], sem_a).wait()

    _idx(0)
    _tab(0)
    _idx(1)
    _tab(1)
    _wait_tab(0)
    _add(0)

    def chunk(k, carry):
        @pl.when(k + 2 < NCH)
        def _():
            _idx(k + 2)

        @pl.when(k + 1 < NCH)
        def _():
            _wait_tab(k + 1)
            _add(k + 1)

        _wait_add(k)
        p = k % 2

        def rrow(r, cc):
            for j in range(H // 16):
                sl = (p, r, pl.ds(j * 16, 16))
                buf[sl] = jnp.maximum(buf[sl], 0.0)
            return cc

        lax.fori_loop(0, CH, rrow, 0)
        pltpu.sync_copy(buf.at[p], aggr.at[idx3.at[k % 3, 1]], add=True)

        @pl.when(k + 2 < NCH)
        def _():
            _tab(k + 2)

        return carry

    lax.fori_loop(0, NCH, chunk, 0)
    plsc.subcore_barrier()
    pltpu.sync_copy(aggr.at[pl.ds(s * RPT, RPT)],
                    out_hbm.at[c, pl.ds(s * RPT, RPT)])


RB = 512
NBLK = NPAD // RB


def _cat(ref):
    b = ref[...]
    return jnp.concatenate([b[0], b[1]], axis=-1)


def _rowmask(i):
    rows = i * RB + lax.broadcasted_iota(jnp.int32, (RB, 1), 0)
    return rows < N


def _store_halves(ref, v):
    ref[0] = v[:, :H]
    ref[1] = v[:, H:]


def _masked_stats(i, st_ref, t):
    tm = jnp.where(_rowmask(i), t, 0.0)
    st = jnp.stack([jnp.sum(tm, axis=0), jnp.sum(tm * tm, axis=0)])

    @pl.when(i == 0)
    def _():
        st_ref[...] = st

    @pl.when(i > 0)
    def _():
        st_ref[...] += st


def _c1_body(eps_ref, hin_ref, aggr_ref, w_ref, b_ref, t1_ref, st_ref):
    i = pl.program_id(0)
    z = eps_ref[0, 0] * _cat(hin_ref) + _cat(aggr_ref)
    t1 = jnp.dot(z, w_ref[...], preferred_element_type=jnp.float32) + b_ref[...]
    _store_halves(t1_ref, t1)
    _masked_stats(i, st_ref, t1)


def _bn_from_stats(st_ref, t, g_ref, b_ref):
    mean = st_ref[0:1, :] / N
    var = st_ref[1:2, :] / N - mean * mean
    inv = lax.rsqrt(var + 1e-5)
    return (t - mean) * inv * g_ref[...] + b_ref[...]


def _c2_body(t1_ref, st_ref, g_ref, bb_ref, w_ref, b_ref, t2_ref, st2_ref):
    i = pl.program_id(0)
    y = jax.nn.relu(_bn_from_stats(st_ref, _cat(t1_ref), g_ref, bb_ref))
    t2 = jnp.dot(y, w_ref[...], preferred_element_type=jnp.float32) + b_ref[...]
    _store_halves(t2_ref, t2)
    _masked_stats(i, st2_ref, t2)


def _c3_body(t2_ref, st_ref, g_ref, bb_ref, hin_ref, vnn_ref, batch_ref,
             out_ref, *, last):
    h = _bn_from_stats(st_ref, _cat(t2_ref), g_ref, bb_ref)
    if not last:
        h = jax.nn.relu(h)
    h = h + _cat(hin_ref)
    if not last:
        onehot = (batch_ref[0][:, None]
                  == lax.broadcasted_iota(jnp.int32, (1, G), 1)
                  ).astype(jnp.float32)
        h = h + jnp.dot(onehot, vnn_ref[...],
                        preferred_element_type=jnp.float32)
    _store_halves(out_ref, h)


def _pool_body(hin_ref, batch_ref, out_ref):
    i = pl.program_id(0)
    onehot_t = (lax.broadcasted_iota(jnp.int32, (G, 1), 0)
                == batch_ref[0][None, :]).astype(jnp.float32)
    p = jnp.dot(onehot_t, _cat(hin_ref), preferred_element_type=jnp.float32)

    @pl.when(i == 0)
    def _():
        out_ref[...] = p

    @pl.when(i > 0)
    def _():
        out_ref[...] += p


def _bn_small(t, g, b):
    m = jnp.mean(t, axis=0, keepdims=True)
    v = jnp.mean(t * t, axis=0, keepdims=True) - m * m
    return (t - m) * lax.rsqrt(v + 1e-5) * g + b


def _vnmlp_body(pool_ref, vn_ref, w1_ref, b1_ref, g1_ref, bb1_ref,
                w2_ref, b2_ref, g2_ref, bb2_ref, out_ref):
    vt = pool_ref[...] + vn_ref[...]
    t = jnp.dot(vt, w1_ref[...], preferred_element_type=jnp.float32) + b1_ref[...]
    t = jax.nn.relu(_bn_small(t, g1_ref[...], bb1_ref[...]))
    t = jnp.dot(t, w2_ref[...], preferred_element_type=jnp.float32) + b2_ref[...]
    t = jax.nn.relu(_bn_small(t, g2_ref[...], bb2_ref[...]))
    out_ref[...] = vn_ref[...] + t


_hspec = pl.BlockSpec((2, RB, H), lambda i: (0, i, 0))
_wspec = pl.BlockSpec((D, D), lambda i: (0, 0))
_bspec = pl.BlockSpec((1, D), lambda i: (0, 0))
_stspec = pl.BlockSpec((2, D), lambda i: (0, 0))
_batchspec = pl.BlockSpec((1, RB), lambda i: (0, i))
_h_sds = jax.ShapeDtypeStruct((2, NPAD, H), jnp.float32)
_st_sds = jax.ShapeDtypeStruct((2, D), jnp.float32)
_g_sds = jax.ShapeDtypeStruct((G, D), jnp.float32)
_gspec = pl.BlockSpec((G, D), lambda i: (0, 0))


def _c1(eps, hin, aggr, w, b):
    return pl.pallas_call(
        _c1_body,
        grid=(NBLK,),
        in_specs=[pl.BlockSpec((1, 1), lambda i: (0, 0)),
                  _hspec, _hspec, _wspec, _bspec],
        out_specs=[_hspec, _stspec],
        out_shape=[_h_sds, _st_sds],
    )(eps, hin, aggr, w, b)


def _c2(t1, st, g, bb, w, b):
    return pl.pallas_call(
        _c2_body,
        grid=(NBLK,),
        in_specs=[_hspec, _stspec, _bspec, _bspec, _wspec, _bspec],
        out_specs=[_hspec, _stspec],
        out_shape=[_h_sds, _st_sds],
    )(t1, st, g, bb, w, b)


def _c3(t2, st, g, bb, hin, vnn, batchp, last):
    if last:
        return pl.pallas_call(
            functools.partial(
                lambda a, b_, c_, d_, e_, f_, o: _c3_body(
                    a, b_, c_, d_, e_, None, f_, o, last=True)),
            grid=(NBLK,),
            in_specs=[_hspec, _stspec, _bspec, _bspec, _hspec, _batchspec],
            out_specs=_hspec,
            out_shape=_h_sds,
        )(t2, st, g, bb, hin, batchp)
    return pl.pallas_call(
        functools.partial(_c3_body, last=False),
        grid=(NBLK,),
        in_specs=[_hspec, _stspec, _bspec, _bspec, _hspec, _gspec, _batchspec],
        out_specs=_hspec,
        out_shape=_h_sds,
    )(t2, st, g, bb, hin, vnn, batchp)


def _pool(hin, batchp):
    return pl.pallas_call(
        _pool_body,
        grid=(NBLK,),
        in_specs=[_hspec, _batchspec],
        out_specs=_gspec,
        out_shape=_g_sds,
    )(hin, batchp)


def _vnmlp(pool, vn, w1, b1, g1, bb1, w2, b2, g2, bb2):
    one = pl.BlockSpec((G, D), lambda: (0, 0))
    bs = pl.BlockSpec((1, D), lambda: (0, 0))
    ws = pl.BlockSpec((D, D), lambda: (0, 0))
    return pl.pallas_call(
        _vnmlp_body,
        in_specs=[one, one, ws, bs, bs, bs, ws, bs, bs, bs],
        out_specs=one,
        out_shape=_g_sds,
    )(pool, vn, w1, b1, g1, bb1, w2, b2, g2, bb2)


def kernel(x, edge_index, edge_attr, batch, atom_emb, vn_emb, conv_eps,
           bond_emb, conv_W1, conv_b1, conv_bn_g, conv_bn_b, conv_W2, conv_b2,
           outer_bn_g, outer_bn_b, vn_W1, vn_b1, vn_bn1_g, vn_bn1_b,
           vn_W2, vn_b2, vn_bn2_g, vn_bn2_b):
    f32 = jnp.float32
    i32 = jnp.int32

    xT = jnp.pad(x.astype(i32).T, ((0, 0), (0, NPAD - N)))
    xg = xT.reshape(NA, NT, AC, CHA).transpose(1, 0, 2, 3)
    src = edge_index[0].astype(i32)
    dst = edge_index[1].astype(i32)
    code = (edge_attr[:, 0] * (VB * VB) + edge_attr[:, 1] * VB
            + edge_attr[:, 2]).astype(i32)
    eg = jnp.stack([src, dst, code]).reshape(3, NT, NCH, CH).transpose(1, 2, 0, 3)
    batchp = jnp.pad(batch.astype(i32), (0, NPAD - N),
                     constant_values=G).reshape(1, NPAD)
    zrows = jnp.zeros((RPT, H), f32)

    atab = atom_emb.at[0].add(vn_emb[0][None, :])
    atabs = jnp.stack([atab[:, :, :H], atab[:, :, H:]])
    et = (bond_emb[:, 0][:, :, None, None, :]
          + bond_emb[:, 1][:, None, :, None, :]
          + bond_emb[:, 2][:, None, None, :, :]).reshape(L, VB ** NB, D)
    ets = jnp.stack([et[..., :H], et[..., H:]], axis=1)

    vn = jnp.broadcast_to(vn_emb[0], (G, D)).astype(f32)
    b1r = conv_b1.reshape(L, 1, D)
    b2r = conv_b2.reshape(L, 1, D)

    hin = _atom_sc(xg, atabs)
    hout = None
    for l in range(L):
        vn_next = None
        if l < L - 1:
            pool = _pool(hin, batchp)
            vn_next = _vnmlp(pool, vn,
                             vn_W1[l], vn_b1[l].reshape(1, D),
                             vn_bn1_g[l].reshape(1, D), vn_bn1_b[l].reshape(1, D),
                             vn_W2[l], vn_b2[l].reshape(1, D),
                             vn_bn2_g[l].reshape(1, D), vn_bn2_b[l].reshape(1, D))
        aggr = _edge_sc(hin, eg, ets[l], zrows)
        eps = (1.0 + conv_eps[l]).reshape(1, 1)
        t1, st1 = _c1(eps, hin, aggr, conv_W1[l], b1r[l])
        t2, st2 = _c2(t1, st1, conv_bn_g[l].reshape(1, D),
                      conv_bn_b[l].reshape(1, D), conv_W2[l], b2r[l])
        if l < L - 1:
            hin = _c3(t2, st2, outer_bn_g[l].reshape(1, D),
                      outer_bn_b[l].reshape(1, D), hin, vn_next, batchp,
                      last=False)
            vn = vn_next
        else:
            hout = _c3(t2, st2, outer_bn_g[l].reshape(1, D),
                       outer_bn_b[l].reshape(1, D), hin, None, batchp,
                       last=True)
    return jnp.concatenate([hout[0][:N], hout[1][:N]], axis=1)


# speedup vs baseline: 1.1718x; 1.1718x over previous
import functools

import jax
import jax.numpy as jnp
from jax import lax
from jax.experimental import pallas as pl
from jax.experimental.pallas import tpu as pltpu
from jax.experimental.pallas import tpu_sc as plsc

N = 10000
E = 160000
D = 256
L = 3
G = 64
NA, VA, NB, VB = 9, 119, 3, 6

H = 128
NT = 16
NPAD = 10240
RPT = NPAD // NT
EPT = E // NT
EPTP = 10240
CH = 128
NCH = EPTP // CH
AC = 8
CHA = RPT // AC

_mesh = plsc.VectorSubcoreMesh(core_axis_name="c", subcore_axis_name="s")


@functools.partial(
    pl.kernel,
    out_type=jax.ShapeDtypeStruct((2, NPAD, H), jnp.float32),
    mesh=_mesh,
    scratch_types=[
        pltpu.VMEM((NA, AC, CHA), jnp.int32),
        pltpu.VMEM((RPT, H), jnp.float32),
        pltpu.SemaphoreType.DMA,
        pltpu.SemaphoreType.DMA,
    ],
)
def _atom_sc(xg_hbm, tab_hbm, out_hbm, idx_v, buf, sem_t, sem_a):
    c = lax.axis_index("c")
    s = lax.axis_index("s")
    pltpu.sync_copy(xg_hbm.at[s], idx_v)

    def _f0(k):
        pltpu.async_copy(
            tab_hbm.at[c, 0].at[idx_v.at[0, k]],
            buf.at[pl.ds(k * CHA, CHA)], sem_t)

    _f0(0)

    def chunk(k, carry):
        pltpu.make_async_copy(
            tab_hbm.at[c, 0].at[idx_v.at[0, k]],
            buf.at[pl.ds(k * CHA, CHA)], sem_t).wait()

        @pl.when(k + 1 < AC)
        def _():
            _f0(k + 1)

        for f in range(1, NA):
            pltpu.async_copy(
                tab_hbm.at[c, f].at[idx_v.at[f, k]],
                buf.at[pl.ds(k * CHA, CHA)], sem_a, add=True)
        return carry

    lax.fori_loop(0, AC, chunk, 0)

    def drain(k, carry):
        pltpu.make_async_copy(
            tab_hbm.at[c, 1].at[idx_v.at[1, 0]],
            buf.at[pl.ds(0, CHA)], sem_a).wait()
        return carry

    lax.fori_loop(0, AC * (NA - 1), drain, 0)
    pltpu.sync_copy(buf, out_hbm.at[c, pl.ds(s * RPT, RPT)])


@functools.partial(
    pl.kernel,
    out_type=jax.ShapeDtypeStruct((2, NPAD, H), jnp.float32),
    mesh=_mesh,
    scratch_types=[
        pltpu.VMEM((2, 3, CH), jnp.int32),
        pltpu.VMEM((2, CH, H), jnp.float32),
        pltpu.VMEM_SHARED((NPAD, H), jnp.float32),
        pltpu.SemaphoreType.DMA,
        pltpu.SemaphoreType.DMA,
    ],
)
def _edge_sc(hin_hbm, eg_hbm, tab_hbm, zrows_hbm, out_hbm,
             idx3, buf, aggr, sem_t, sem_a):
    c = lax.axis_index("c")
    s = lax.axis_index("s")
    pltpu.sync_copy(zrows_hbm, aggr.at[pl.ds(s * RPT, RPT)])
    plsc.subcore_barrier()

    def _relu(p):
        def rrow(r, cc):
            for j in range(H // 16):
                sl = (p, r, pl.ds(j * 16, 16))
                buf[sl] = jnp.maximum(buf[sl], 0.0)
            return cc

        lax.fori_loop(0, CH, rrow, 0)

    def body(i, carry):
        pltpu.sync_copy(eg_hbm.at[s, pl.ds(2 * i, 2)], idx3)
        dt0 = pltpu.async_copy(tab_hbm.at[c].at[idx3.at[0, 2]],
                               buf.at[0], sem_t)
        dt1 = pltpu.async_copy(tab_hbm.at[c].at[idx3.at[1, 2]],
                               buf.at[1], sem_t)
        dt0.wait()
        da0 = pltpu.async_copy(hin_hbm.at[c].at[idx3.at[0, 0]],
                               buf.at[0], sem_a, add=True)
        dt1.wait()
        da1 = pltpu.async_copy(hin_hbm.at[c].at[idx3.at[1, 0]],
                               buf.at[1], sem_a, add=True)
        da0.wait()
        _relu(0)
        pltpu.sync_copy(buf.at[0], aggr.at[idx3.at[0, 1]], add=True)
        da1.wait()
        _relu(1)
        pltpu.sync_copy(buf.at[1], aggr.at[idx3.at[1, 1]], add=True)
        return carry

    lax.fori_loop(0, NCH // 2, body, 0)
    plsc.subcore_barrier()
    pltpu.sync_copy(aggr.at[pl.ds(s * RPT, RPT)],
                    out_hbm.at[c, pl.ds(s * RPT, RPT)])


RB = 512
NBLK = NPAD // RB


def _cat(ref):
    b = ref[...]
    return jnp.concatenate([b[0], b[1]], axis=-1)


def _rowmask(i):
    rows = i * RB + lax.broadcasted_iota(jnp.int32, (RB, 1), 0)
    return rows < N


def _store_halves(ref, v):
    ref[0] = v[:, :H]
    ref[1] = v[:, H:]


def _masked_stats(i, st_ref, t):
    tm = jnp.where(_rowmask(i), t, 0.0)
    st = jnp.stack([jnp.sum(tm, axis=0), jnp.sum(tm * tm, axis=0)])

    @pl.when(i == 0)
    def _():
        st_ref[...] = st

    @pl.when(i > 0)
    def _():
        st_ref[...] += st


def _c1_body(eps_ref, hin_ref, aggr_ref, w_ref, b_ref, t1_ref, st_ref):
    i = pl.program_id(0)
    z = eps_ref[0, 0] * _cat(hin_ref) + _cat(aggr_ref)
    t1 = jnp.dot(z, w_ref[...], preferred_element_type=jnp.float32) + b_ref[...]
    _store_halves(t1_ref, t1)
    _masked_stats(i, st_ref, t1)


def _bn_from_stats(st_ref, t, g_ref, b_ref):
    mean = st_ref[0:1, :] / N
    var = st_ref[1:2, :] / N - mean * mean
    inv = lax.rsqrt(var + 1e-5)
    return (t - mean) * inv * g_ref[...] + b_ref[...]


def _c2_body(t1_ref, st_ref, g_ref, bb_ref, w_ref, b_ref, t2_ref, st2_ref):
    i = pl.program_id(0)
    y = jax.nn.relu(_bn_from_stats(st_ref, _cat(t1_ref), g_ref, bb_ref))
    t2 = jnp.dot(y, w_ref[...], preferred_element_type=jnp.float32) + b_ref[...]
    _store_halves(t2_ref, t2)
    _masked_stats(i, st2_ref, t2)


def _c3_body(t2_ref, st_ref, g_ref, bb_ref, hin_ref, vnn_ref, batch_ref,
             out_ref, *, last):
    h = _bn_from_stats(st_ref, _cat(t2_ref), g_ref, bb_ref)
    if not last:
        h = jax.nn.relu(h)
    h = h + _cat(hin_ref)
    if not last:
        onehot = (batch_ref[0][:, None]
                  == lax.broadcasted_iota(jnp.int32, (1, G), 1)
                  ).astype(jnp.float32)
        h = h + jnp.dot(onehot, vnn_ref[...],
                        preferred_element_type=jnp.float32)
    _store_halves(out_ref, h)


def _pool_body(hin_ref, batch_ref, out_ref):
    i = pl.program_id(0)
    onehot_t = (lax.broadcasted_iota(jnp.int32, (G, 1), 0)
                == batch_ref[0][None, :]).astype(jnp.float32)
    p = jnp.dot(onehot_t, _cat(hin_ref), preferred_element_type=jnp.float32)

    @pl.when(i == 0)
    def _():
        out_ref[...] = p

    @pl.when(i > 0)
    def _():
        out_ref[...] += p


def _bn_small(t, g, b):
    m = jnp.mean(t, axis=0, keepdims=True)
    v = jnp.mean(t * t, axis=0, keepdims=True) - m * m
    return (t - m) * lax.rsqrt(v + 1e-5) * g + b


def _vnmlp_body(pool_ref, vn_ref, w1_ref, b1_ref, g1_ref, bb1_ref,
                w2_ref, b2_ref, g2_ref, bb2_ref, out_ref):
    vt = pool_ref[...] + vn_ref[...]
    t = jnp.dot(vt, w1_ref[...], preferred_element_type=jnp.float32) + b1_ref[...]
    t = jax.nn.relu(_bn_small(t, g1_ref[...], bb1_ref[...]))
    t = jnp.dot(t, w2_ref[...], preferred_element_type=jnp.float32) + b2_ref[...]
    t = jax.nn.relu(_bn_small(t, g2_ref[...], bb2_ref[...]))
    out_ref[...] = vn_ref[...] + t


_hspec = pl.BlockSpec((2, RB, H), lambda i: (0, i, 0))
_wspec = pl.BlockSpec((D, D), lambda i: (0, 0))
_bspec = pl.BlockSpec((1, D), lambda i: (0, 0))
_stspec = pl.BlockSpec((2, D), lambda i: (0, 0))
_batchspec = pl.BlockSpec((1, RB), lambda i: (0, i))
_h_sds = jax.ShapeDtypeStruct((2, NPAD, H), jnp.float32)
_st_sds = jax.ShapeDtypeStruct((2, D), jnp.float32)
_g_sds = jax.ShapeDtypeStruct((G, D), jnp.float32)
_gspec = pl.BlockSpec((G, D), lambda i: (0, 0))


def _c1(eps, hin, aggr, w, b):
    return pl.pallas_call(
        _c1_body,
        grid=(NBLK,),
        in_specs=[pl.BlockSpec((1, 1), lambda i: (0, 0)),
                  _hspec, _hspec, _wspec, _bspec],
        out_specs=[_hspec, _stspec],
        out_shape=[_h_sds, _st_sds],
    )(eps, hin, aggr, w, b)


def _c2(t1, st, g, bb, w, b):
    return pl.pallas_call(
        _c2_body,
        grid=(NBLK,),
        in_specs=[_hspec, _stspec, _bspec, _bspec, _wspec, _bspec],
        out_specs=[_hspec, _stspec],
        out_shape=[_h_sds, _st_sds],
    )(t1, st, g, bb, w, b)


def _c3(t2, st, g, bb, hin, vnn, batchp, last):
    if last:
        return pl.pallas_call(
            functools.partial(
                lambda a, b_, c_, d_, e_, f_, o: _c3_body(
                    a, b_, c_, d_, e_, None, f_, o, last=True)),
            grid=(NBLK,),
            in_specs=[_hspec, _stspec, _bspec, _bspec, _hspec, _batchspec],
            out_specs=_hspec,
            out_shape=_h_sds,
        )(t2, st, g, bb, hin, batchp)
    return pl.pallas_call(
        functools.partial(_c3_body, last=False),
        grid=(NBLK,),
        in_specs=[_hspec, _stspec, _bspec, _bspec, _hspec, _gspec, _batchspec],
        out_specs=_hspec,
        out_shape=_h_sds,
    )(t2, st, g, bb, hin, vnn, batchp)


def _pool(hin, batchp):
    return pl.pallas_call(
        _pool_body,
        grid=(NBLK,),
        in_specs=[_hspec, _batchspec],
        out_specs=_gspec,
        out_shape=_g_sds,
    )(hin, batchp)


def _vnmlp(pool, vn, w1, b1, g1, bb1, w2, b2, g2, bb2):
    one = pl.BlockSpec((G, D), lambda: (0, 0))
    bs = pl.BlockSpec((1, D), lambda: (0, 0))
    ws = pl.BlockSpec((D, D), lambda: (0, 0))
    return pl.pallas_call(
        _vnmlp_body,
        in_specs=[one, one, ws, bs, bs, bs, ws, bs, bs, bs],
        out_specs=one,
        out_shape=_g_sds,
    )(pool, vn, w1, b1, g1, bb1, w2, b2, g2, bb2)


def kernel(x, edge_index, edge_attr, batch, atom_emb, vn_emb, conv_eps,
           bond_emb, conv_W1, conv_b1, conv_bn_g, conv_bn_b, conv_W2, conv_b2,
           outer_bn_g, outer_bn_b, vn_W1, vn_b1, vn_bn1_g, vn_bn1_b,
           vn_W2, vn_b2, vn_bn2_g, vn_bn2_b):
    f32 = jnp.float32
    i32 = jnp.int32

    xT = jnp.pad(x.astype(i32).T, ((0, 0), (0, NPAD - N)))
    xg = xT.reshape(NA, NT, AC, CHA).transpose(1, 0, 2, 3)
    src = edge_index[0].astype(i32)
    dst = edge_index[1].astype(i32)
    code = (edge_attr[:, 0] * (VB * VB) + edge_attr[:, 1] * VB
            + edge_attr[:, 2]).astype(i32)
    pad = ((0, 0), (0, EPTP - EPT))
    srcp = jnp.pad(src.reshape(NT, EPT), pad)
    dstp = jnp.pad(dst.reshape(NT, EPT), pad, constant_values=NPAD - 1)
    codep = jnp.pad(code.reshape(NT, EPT), pad)
    eg = jnp.stack([srcp, dstp, codep], 1).reshape(NT, 3, NCH, CH)
    eg = eg.transpose(0, 2, 1, 3)
    batchp = jnp.pad(batch.astype(i32), (0, NPAD - N),
                     constant_values=G).reshape(1, NPAD)
    zrows = jnp.zeros((RPT, H), f32)

    atab = atom_emb.at[0].add(vn_emb[0][None, :])
    atabs = jnp.stack([atab[:, :, :H], atab[:, :, H:]])
    et = (bond_emb[:, 0][:, :, None, None, :]
          + bond_emb[:, 1][:, None, :, None, :]
          + bond_emb[:, 2][:, None, None, :, :]).reshape(L, VB ** NB, D)
    ets = jnp.stack([et[..., :H], et[..., H:]], axis=1)

    vn = jnp.broadcast_to(vn_emb[0], (G, D)).astype(f32)
    b1r = conv_b1.reshape(L, 1, D)
    b2r = conv_b2.reshape(L, 1, D)

    hin = _atom_sc(xg, atabs)
    hout = None
    for l in range(L):
        vn_next = None
        if l < L - 1:
            pool = _pool(hin, batchp)
            vn_next = _vnmlp(pool, vn,
                             vn_W1[l], vn_b1[l].reshape(1, D),
                             vn_bn1_g[l].reshape(1, D), vn_bn1_b[l].reshape(1, D),
                             vn_W2[l], vn_b2[l].reshape(1, D),
                             vn_bn2_g[l].reshape(1, D), vn_bn2_b[l].reshape(1, D))
        aggr = _edge_sc(hin, eg, ets[l], zrows)
        eps = (1.0 + conv_eps[l]).reshape(1, 1)
        t1, st1 = _c1(eps, hin, aggr, conv_W1[l], b1r[l])
        t2, st2 = _c2(t1, st1, conv_bn_g[l].reshape(1, D),
                      conv_bn_b[l].reshape(1, D), conv_W2[l], b2r[l])
        if l < L - 1:
            hin = _c3(t2, st2, outer_bn_g[l].reshape(1, D),
                      outer_bn_b[l].reshape(1, D), hin, vn_next, batchp,
                      last=False)
            vn = vn_next
        else:
            hout = _c3(t2, st2, outer_bn_g[l].reshape(1, D),
                       outer_bn_b[l].reshape(1, D), hin, None, batchp,
                       last=True)
    return jnp.concatenate([hout[0][:N], hout[1][:N]], axis=1)


# speedup vs baseline: 1.2215x; 1.0424x over previous
import functools

import jax
import jax.numpy as jnp
from jax import lax
from jax.experimental import pallas as pl
from jax.experimental.pallas import tpu as pltpu
from jax.experimental.pallas import tpu_sc as plsc

N = 10000
E = 160000
D = 256
L = 3
G = 64
NA, VA, NB, VB = 9, 119, 3, 6

H = 128
NT = 16
NPAD = 10240
RPT = NPAD // NT
EPT = E // NT
EPTP = 10240
CH = 64
NCH = EPTP // CH
GRP = 8
NGRP = NCH // GRP
AC = 8
CHA = RPT // AC

_mesh = plsc.VectorSubcoreMesh(core_axis_name="c", subcore_axis_name="s")


@functools.partial(
    pl.kernel,
    out_type=jax.ShapeDtypeStruct((2, NPAD, H), jnp.float32),
    mesh=_mesh,
    scratch_types=[
        pltpu.VMEM((NA, AC, CHA), jnp.int32),
        pltpu.VMEM((RPT, H), jnp.float32),
        pltpu.SemaphoreType.DMA,
        pltpu.SemaphoreType.DMA,
    ],
)
def _atom_sc(xg_hbm, tab_hbm, out_hbm, idx_v, buf, sem_t, sem_a):
    c = lax.axis_index("c")
    s = lax.axis_index("s")
    pltpu.sync_copy(xg_hbm.at[s], idx_v)

    def _f0(k):
        pltpu.async_copy(
            tab_hbm.at[c, 0].at[idx_v.at[0, k]],
            buf.at[pl.ds(k * CHA, CHA)], sem_t)

    _f0(0)

    def chunk(k, carry):
        pltpu.make_async_copy(
            tab_hbm.at[c, 0].at[idx_v.at[0, k]],
            buf.at[pl.ds(k * CHA, CHA)], sem_t).wait()

        @pl.when(k + 1 < AC)
        def _():
            _f0(k + 1)

        for f in range(1, NA):
            pltpu.async_copy(
                tab_hbm.at[c, f].at[idx_v.at[f, k]],
                buf.at[pl.ds(k * CHA, CHA)], sem_a, add=True)
        return carry

    lax.fori_loop(0, AC, chunk, 0)

    def drain(k, carry):
        pltpu.make_async_copy(
            tab_hbm.at[c, 1].at[idx_v.at[1, 0]],
            buf.at[pl.ds(0, CHA)], sem_a).wait()
        return carry

    lax.fori_loop(0, AC * (NA - 1), drain, 0)
    pltpu.sync_copy(buf, out_hbm.at[c, pl.ds(s * RPT, RPT)])


@functools.partial(
    pl.kernel,
    out_type=jax.ShapeDtypeStruct((2, NPAD, H), jnp.float32),
    mesh=_mesh,
    scratch_types=[
        pltpu.VMEM((2, GRP * CH), jnp.int32),
        pltpu.VMEM((GRP, CH), jnp.int32),
        pltpu.VMEM_SHARED((216, H), jnp.float32),
        pltpu.VMEM((2, CH, H), jnp.float32),
        pltpu.VMEM_SHARED((NPAD, H), jnp.float32),
        pltpu.SemaphoreType.DMA,
        pltpu.SemaphoreType.DMA,
    ],
)
def _edge_sc(hin_hbm, ega_hbm, egb_hbm, tab_hbm, zrows_hbm, out_hbm,
             idxa, idxd, tabv, buf, aggr, sem_t, sem_a):
    c = lax.axis_index("c")
    s = lax.axis_index("s")
    @pl.when(s == 0)
    def _():
        pltpu.sync_copy(tab_hbm.at[c], tabv)

    pltpu.sync_copy(zrows_hbm, aggr.at[pl.ds(s * RPT, RPT)])
    plsc.subcore_barrier()

    def _relu(p):
        def rrow(r, cc):
            for j in range(H // 16):
                sl = (p, r, pl.ds(j * 16, 16))
                buf[sl] = jnp.maximum(buf[sl], 0.0)
            return cc

        lax.fori_loop(0, CH, rrow, 0)

    def _tab(u):
        return pltpu.async_copy(
            tabv.at[idxa.at[1, pl.ds(u * CH, CH)]], buf.at[u % 2], sem_t)

    def _gather(u):
        return pltpu.async_copy(
            hin_hbm.at[c].at[idxa.at[0, pl.ds(u * CH, CH)]],
            buf.at[u % 2], sem_a, add=True)

    def body(i, carry):
        pltpu.sync_copy(ega_hbm.at[s, i], idxa)
        pltpu.sync_copy(egb_hbm.at[s, i], idxd)
        dt = [_tab(0), _tab(1)]
        da = []
        dt[0].wait()
        da.append(_gather(0))
        dt[1].wait()
        da.append(_gather(1))
        for u in range(GRP):
            da[u].wait()
            _relu(u % 2)
            pltpu.sync_copy(buf.at[u % 2], aggr.at[idxd.at[u]], add=True)
            if u + 2 < GRP:
                dt.append(_tab(u + 2))
                dt[u + 2].wait()
                da.append(_gather(u + 2))
        return carry

    lax.fori_loop(0, NGRP, body, 0)
    plsc.subcore_barrier()
    pltpu.sync_copy(aggr.at[pl.ds(s * RPT, RPT)],
                    out_hbm.at[c, pl.ds(s * RPT, RPT)])


RB = 512
NBLK = NPAD // RB


def _cat(ref):
    b = ref[...]
    return jnp.concatenate([b[0], b[1]], axis=-1)


def _rowmask(i):
    rows = i * RB + lax.broadcasted_iota(jnp.int32, (RB, 1), 0)
    return rows < N


def _store_halves(ref, v):
    ref[0] = v[:, :H]
    ref[1] = v[:, H:]


def _masked_stats(i, st_ref, t):
    tm = jnp.where(_rowmask(i), t, 0.0)
    st = jnp.stack([jnp.sum(tm, axis=0), jnp.sum(tm * tm, axis=0)])

    @pl.when(i == 0)
    def _():
        st_ref[...] = st

    @pl.when(i > 0)
    def _():
        st_ref[...] += st


def _c1_body(eps_ref, hin_ref, aggr_ref, w_ref, b_ref, t1_ref, st_ref):
    i = pl.program_id(0)
    z = eps_ref[0, 0] * _cat(hin_ref) + _cat(aggr_ref)
    t1 = jnp.dot(z, w_ref[...], preferred_element_type=jnp.float32) + b_ref[...]
    _store_halves(t1_ref, t1)
    _masked_stats(i, st_ref, t1)


def _bn_from_stats(st_ref, t, g_ref, b_ref):
    mean = st_ref[0:1, :] / N
    var = st_ref[1:2, :] / N - mean * mean
    inv = lax.rsqrt(var + 1e-5)
    return (t - mean) * inv * g_ref[...] + b_ref[...]


def _c2_body(t1_ref, st_ref, g_ref, bb_ref, w_ref, b_ref, t2_ref, st2_ref):
    i = pl.program_id(0)
    y = jax.nn.relu(_bn_from_stats(st_ref, _cat(t1_ref), g_ref, bb_ref))
    t2 = jnp.dot(y, w_ref[...], preferred_element_type=jnp.float32) + b_ref[...]
    _store_halves(t2_ref, t2)
    _masked_stats(i, st2_ref, t2)


def _c3_body(t2_ref, st_ref, g_ref, bb_ref, hin_ref, vnn_ref, batch_ref,
             out_ref, *, last):
    h = _bn_from_stats(st_ref, _cat(t2_ref), g_ref, bb_ref)
    if not last:
        h = jax.nn.relu(h)
    h = h + _cat(hin_ref)
    if not last:
        onehot = (batch_ref[0][:, None]
                  == lax.broadcasted_iota(jnp.int32, (1, G), 1)
                  ).astype(jnp.float32)
        h = h + jnp.dot(onehot, vnn_ref[...],
                        preferred_element_type=jnp.float32)
    _store_halves(out_ref, h)


def _pool_body(hin_ref, batch_ref, out_ref):
    i = pl.program_id(0)
    onehot_t = (lax.broadcasted_iota(jnp.int32, (G, 1), 0)
                == batch_ref[0][None, :]).astype(jnp.float32)
    p = jnp.dot(onehot_t, _cat(hin_ref), preferred_element_type=jnp.float32)

    @pl.when(i == 0)
    def _():
        out_ref[...] = p

    @pl.when(i > 0)
    def _():
        out_ref[...] += p


def _bn_small(t, g, b):
    m = jnp.mean(t, axis=0, keepdims=True)
    v = jnp.mean(t * t, axis=0, keepdims=True) - m * m
    return (t - m) * lax.rsqrt(v + 1e-5) * g + b


def _vnmlp_body(pool_ref, vn_ref, w1_ref, b1_ref, g1_ref, bb1_ref,
                w2_ref, b2_ref, g2_ref, bb2_ref, out_ref):
    vt = pool_ref[...] + vn_ref[...]
    t = jnp.dot(vt, w1_ref[...], preferred_element_type=jnp.float32) + b1_ref[...]
    t = jax.nn.relu(_bn_small(t, g1_ref[...], bb1_ref[...]))
    t = jnp.dot(t, w2_ref[...], preferred_element_type=jnp.float32) + b2_ref[...]
    t = jax.nn.relu(_bn_small(t, g2_ref[...], bb2_ref[...]))
    out_ref[...] = vn_ref[...] + t


_hspec = pl.BlockSpec((2, RB, H), lambda i: (0, i, 0))
_wspec = pl.BlockSpec((D, D), lambda i: (0, 0))
_bspec = pl.BlockSpec((1, D), lambda i: (0, 0))
_stspec = pl.BlockSpec((2, D), lambda i: (0, 0))
_batchspec = pl.BlockSpec((1, RB), lambda i: (0, i))
_h_sds = jax.ShapeDtypeStruct((2, NPAD, H), jnp.float32)
_st_sds = jax.ShapeDtypeStruct((2, D), jnp.float32)
_g_sds = jax.ShapeDtypeStruct((G, D), jnp.float32)
_gspec = pl.BlockSpec((G, D), lambda i: (0, 0))


def _c1(eps, hin, aggr, w, b):
    return pl.pallas_call(
        _c1_body,
        grid=(NBLK,),
        in_specs=[pl.BlockSpec((1, 1), lambda i: (0, 0)),
                  _hspec, _hspec, _wspec, _bspec],
        out_specs=[_hspec, _stspec],
        out_shape=[_h_sds, _st_sds],
    )(eps, hin, aggr, w, b)


def _c2(t1, st, g, bb, w, b):
    return pl.pallas_call(
        _c2_body,
        grid=(NBLK,),
        in_specs=[_hspec, _stspec, _bspec, _bspec, _wspec, _bspec],
        out_specs=[_hspec, _stspec],
        out_shape=[_h_sds, _st_sds],
    )(t1, st, g, bb, w, b)


def _c3(t2, st, g, bb, hin, vnn, batchp, last):
    if last:
        return pl.pallas_call(
            functools.partial(
                lambda a, b_, c_, d_, e_, f_, o: _c3_body(
                    a, b_, c_, d_, e_, None, f_, o, last=True)),
            grid=(NBLK,),
            in_specs=[_hspec, _stspec, _bspec, _bspec, _hspec, _batchspec],
            out_specs=_hspec,
            out_shape=_h_sds,
        )(t2, st, g, bb, hin, batchp)
    return pl.pallas_call(
        functools.partial(_c3_body, last=False),
        grid=(NBLK,),
        in_specs=[_hspec, _stspec, _bspec, _bspec, _hspec, _gspec, _batchspec],
        out_specs=_hspec,
        out_shape=_h_sds,
    )(t2, st, g, bb, hin, vnn, batchp)


def _pool(hin, batchp):
    return pl.pallas_call(
        _pool_body,
        grid=(NBLK,),
        in_specs=[_hspec, _batchspec],
        out_specs=_gspec,
        out_shape=_g_sds,
    )(hin, batchp)


def _vnmlp(pool, vn, w1, b1, g1, bb1, w2, b2, g2, bb2):
    one = pl.BlockSpec((G, D), lambda: (0, 0))
    bs = pl.BlockSpec((1, D), lambda: (0, 0))
    ws = pl.BlockSpec((D, D), lambda: (0, 0))
    return pl.pallas_call(
        _vnmlp_body,
        in_specs=[one, one, ws, bs, bs, bs, ws, bs, bs, bs],
        out_specs=one,
        out_shape=_g_sds,
    )(pool, vn, w1, b1, g1, bb1, w2, b2, g2, bb2)


def kernel(x, edge_index, edge_attr, batch, atom_emb, vn_emb, conv_eps,
           bond_emb, conv_W1, conv_b1, conv_bn_g, conv_bn_b, conv_W2, conv_b2,
           outer_bn_g, outer_bn_b, vn_W1, vn_b1, vn_bn1_g, vn_bn1_b,
           vn_W2, vn_b2, vn_bn2_g, vn_bn2_b):
    f32 = jnp.float32
    i32 = jnp.int32

    xT = jnp.pad(x.astype(i32).T, ((0, 0), (0, NPAD - N)))
    xg = xT.reshape(NA, NT, AC, CHA).transpose(1, 0, 2, 3)
    src = edge_index[0].astype(i32)
    dst = edge_index[1].astype(i32)
    code = (edge_attr[:, 0] * (VB * VB) + edge_attr[:, 1] * VB
            + edge_attr[:, 2]).astype(i32)
    pad = ((0, 0), (0, EPTP - EPT))
    srcp = jnp.pad(src.reshape(NT, EPT), pad)
    dstp = jnp.pad(dst.reshape(NT, EPT), pad, constant_values=NPAD - 1)
    codep = jnp.pad(code.reshape(NT, EPT), pad)
    ega = jnp.stack([srcp, codep], 1).reshape(NT, 2, NGRP, GRP * CH)
    ega = ega.transpose(0, 2, 1, 3)
    egb = dstp.reshape(NT, NGRP, GRP, CH)
    batchp = jnp.pad(batch.astype(i32), (0, NPAD - N),
                     constant_values=G).reshape(1, NPAD)
    zrows = jnp.zeros((RPT, H), f32)

    atab = atom_emb.at[0].add(vn_emb[0][None, :])
    atabs = jnp.stack([atab[:, :, :H], atab[:, :, H:]])
    et = (bond_emb[:, 0][:, :, None, None, :]
          + bond_emb[:, 1][:, None, :, None, :]
          + bond_emb[:, 2][:, None, None, :, :]).reshape(L, VB ** NB, D)
    ets = jnp.stack([et[..., :H], et[..., H:]], axis=1)

    vn = jnp.broadcast_to(vn_emb[0], (G, D)).astype(f32)
    b1r = conv_b1.reshape(L, 1, D)
    b2r = conv_b2.reshape(L, 1, D)

    hin = _atom_sc(xg, atabs)
    hout = None
    for l in range(L):
        vn_next = None
        if l < L - 1:
            pool = _pool(hin, batchp)
            vn_next = _vnmlp(pool, vn,
                             vn_W1[l], vn_b1[l].reshape(1, D),
                             vn_bn1_g[l].reshape(1, D), vn_bn1_b[l].reshape(1, D),
                             vn_W2[l], vn_b2[l].reshape(1, D),
                             vn_bn2_g[l].reshape(1, D), vn_bn2_b[l].reshape(1, D))
        aggr = _edge_sc(hin, ega, egb, ets[l], zrows)
        eps = (1.0 + conv_eps[l]).reshape(1, 1)
        t1, st1 = _c1(eps, hin, aggr, conv_W1[l], b1r[l])
        t2, st2 = _c2(t1, st1, conv_bn_g[l].reshape(1, D),
                      conv_bn_b[l].reshape(1, D), conv_W2[l], b2r[l])
        if l < L - 1:
            hin = _c3(t2, st2, outer_bn_g[l].reshape(1, D),
                      outer_bn_b[l].reshape(1, D), hin, vn_next, batchp,
                      last=False)
            vn = vn_next
        else:
            hout = _c3(t2, st2, outer_bn_g[l].reshape(1, D),
                       outer_bn_b[l].reshape(1, D), hin, None, batchp,
                       last=True)
    return jnp.concatenate([hout[0][:N], hout[1][:N]], axis=1)


# speedup vs baseline: 1.3936x; 1.1409x over previous
import functools

import jax
import jax.numpy as jnp
from jax import lax
from jax.experimental import pallas as pl
from jax.experimental.pallas import tpu as pltpu
from jax.experimental.pallas import tpu_sc as plsc

N = 10000
E = 160000
D = 256
L = 3
G = 64
NA, VA, NB, VB = 9, 119, 3, 6

H = 128
NT = 16
NPAD = 10240
RPT = NPAD // NT
EPT = E // NT
EPTP = 10240
CH = 128
NCH = EPTP // CH
GRP = 8
NGRP = NCH // GRP
AC = 8
CHA = RPT // AC

_mesh = plsc.VectorSubcoreMesh(core_axis_name="c", subcore_axis_name="s")


@functools.partial(
    pl.kernel,
    out_type=jax.ShapeDtypeStruct((2, NPAD, H), jnp.float32),
    mesh=_mesh,
    scratch_types=[
        pltpu.VMEM((NA, AC, CHA), jnp.int32),
        pltpu.VMEM((RPT, H), jnp.float32),
        pltpu.SemaphoreType.DMA,
        pltpu.SemaphoreType.DMA,
    ],
)
def _atom_sc(xg_hbm, tab_hbm, out_hbm, idx_v, buf, sem_t, sem_a):
    c = lax.axis_index("c")
    s = lax.axis_index("s")
    pltpu.sync_copy(xg_hbm.at[s], idx_v)

    def _f0(k):
        pltpu.async_copy(
            tab_hbm.at[c, 0].at[idx_v.at[0, k]],
            buf.at[pl.ds(k * CHA, CHA)], sem_t)

    _f0(0)

    def chunk(k, carry):
        pltpu.make_async_copy(
            tab_hbm.at[c, 0].at[idx_v.at[0, k]],
            buf.at[pl.ds(k * CHA, CHA)], sem_t).wait()

        @pl.when(k + 1 < AC)
        def _():
            _f0(k + 1)

        for f in range(1, NA):
            pltpu.async_copy(
                tab_hbm.at[c, f].at[idx_v.at[f, k]],
                buf.at[pl.ds(k * CHA, CHA)], sem_a, add=True)
        return carry

    lax.fori_loop(0, AC, chunk, 0)

    def drain(k, carry):
        pltpu.make_async_copy(
            tab_hbm.at[c, 1].at[idx_v.at[1, 0]],
            buf.at[pl.ds(0, CHA)], sem_a).wait()
        return carry

    lax.fori_loop(0, AC * (NA - 1), drain, 0)
    pltpu.sync_copy(buf, out_hbm.at[c, pl.ds(s * RPT, RPT)])


@functools.partial(
    pl.kernel,
    out_type=jax.ShapeDtypeStruct((2, NPAD, H), jnp.float32),
    mesh=_mesh,
    scratch_types=[
        pltpu.VMEM((2, GRP * CH), jnp.int32),
        pltpu.VMEM((GRP, CH), jnp.int32),
        pltpu.VMEM_SHARED((216, H), jnp.float32),
        pltpu.VMEM((2, CH, H), jnp.float32),
        pltpu.VMEM_SHARED((NPAD, H), jnp.float32),
        pltpu.SemaphoreType.DMA,
        pltpu.SemaphoreType.DMA,
    ],
)
def _edge_sc(hin_hbm, ega_hbm, egb_hbm, tab_hbm, zrows_hbm, out_hbm,
             idxa, idxd, tabv, buf, aggr, sem_t, sem_a):
    c = lax.axis_index("c")
    s = lax.axis_index("s")
    @pl.when(s == 0)
    def _():
        pltpu.sync_copy(tab_hbm.at[c], tabv)

    pltpu.sync_copy(zrows_hbm, aggr.at[pl.ds(s * RPT, RPT)])
    plsc.subcore_barrier()

    def _relu(p):
        def rrow(r, cc):
            for j in range(H // 16):
                sl = (p, r, pl.ds(j * 16, 16))
                buf[sl] = jnp.maximum(buf[sl], 0.0)
            return cc

        lax.fori_loop(0, CH, rrow, 0)

    def _tab(u):
        return pltpu.async_copy(
            tabv.at[idxa.at[1, pl.ds(u * CH, CH)]], buf.at[u % 2], sem_t)

    def _gather(u):
        return pltpu.async_copy(
            hin_hbm.at[c].at[idxa.at[0, pl.ds(u * CH, CH)]],
            buf.at[u % 2], sem_a, add=True)

    def body(i, carry):
        pltpu.sync_copy(ega_hbm.at[s, i], idxa)
        pltpu.sync_copy(egb_hbm.at[s, i], idxd)
        dt = [_tab(0), _tab(1)]
        da = []
        dt[0].wait()
        da.append(_gather(0))
        dt[1].wait()
        da.append(_gather(1))
        for u in range(GRP):
            da[u].wait()
            _relu(u % 2)
            pltpu.sync_copy(buf.at[u % 2], aggr.at[idxd.at[u]], add=True)
            if u + 2 < GRP:
                dt.append(_tab(u + 2))
                dt[u + 2].wait()
                da.append(_gather(u + 2))
        return carry

    lax.fori_loop(0, NGRP, body, 0)
    plsc.subcore_barrier()
    pltpu.sync_copy(aggr.at[pl.ds(s * RPT, RPT)],
                    out_hbm.at[c, pl.ds(s * RPT, RPT)])


RB = 512
NBLK = NPAD // RB


def _cat(ref):
    b = ref[...]
    return jnp.concatenate([b[0], b[1]], axis=-1)


def _rowmask(i):
    rows = i * RB + lax.broadcasted_iota(jnp.int32, (RB, 1), 0)
    return rows < N


def _store_halves(ref, v):
    ref[0] = v[:, :H]
    ref[1] = v[:, H:]


def _masked_stats(i, st_ref, t):
    tm = jnp.where(_rowmask(i), t, 0.0)
    st = jnp.stack([jnp.sum(tm, axis=0), jnp.sum(tm * tm, axis=0)])

    @pl.when(i == 0)
    def _():
        st_ref[...] = st

    @pl.when(i > 0)
    def _():
        st_ref[...] += st


def _c1_body(eps_ref, hin_ref, aggr_ref, w_ref, b_ref, t1_ref, st_ref):
    i = pl.program_id(0)
    z = eps_ref[0, 0] * _cat(hin_ref) + _cat(aggr_ref)
    t1 = jnp.dot(z, w_ref[...], preferred_element_type=jnp.float32) + b_ref[...]
    _store_halves(t1_ref, t1)
    _masked_stats(i, st_ref, t1)


def _bn_from_stats(st_ref, t, g_ref, b_ref):
    mean = st_ref[0:1, :] / N
    var = st_ref[1:2, :] / N - mean * mean
    inv = lax.rsqrt(var + 1e-5)
    return (t - mean) * inv * g_ref[...] + b_ref[...]


def _c2_body(t1_ref, st_ref, g_ref, bb_ref, w_ref, b_ref, t2_ref, st2_ref):
    i = pl.program_id(0)
    y = jax.nn.relu(_bn_from_stats(st_ref, _cat(t1_ref), g_ref, bb_ref))
    t2 = jnp.dot(y, w_ref[...], preferred_element_type=jnp.float32) + b_ref[...]
    _store_halves(t2_ref, t2)
    _masked_stats(i, st2_ref, t2)


def _c3_body(t2_ref, st_ref, g_ref, bb_ref, hin_ref, vnn_ref, batch_ref,
             out_ref, *, last):
    h = _bn_from_stats(st_ref, _cat(t2_ref), g_ref, bb_ref)
    if not last:
        h = jax.nn.relu(h)
    h = h + _cat(hin_ref)
    if not last:
        onehot = (batch_ref[0][:, None]
                  == lax.broadcasted_iota(jnp.int32, (1, G), 1)
                  ).astype(jnp.float32)
        h = h + jnp.dot(onehot, vnn_ref[...],
                        preferred_element_type=jnp.float32)
    _store_halves(out_ref, h)


def _pool_body(hin_ref, batch_ref, out_ref):
    i = pl.program_id(0)
    onehot_t = (lax.broadcasted_iota(jnp.int32, (G, 1), 0)
                == batch_ref[0][None, :]).astype(jnp.float32)
    p = jnp.dot(onehot_t, _cat(hin_ref), preferred_element_type=jnp.float32)

    @pl.when(i == 0)
    def _():
        out_ref[...] = p

    @pl.when(i > 0)
    def _():
        out_ref[...] += p


def _bn_small(t, g, b):
    m = jnp.mean(t, axis=0, keepdims=True)
    v = jnp.mean(t * t, axis=0, keepdims=True) - m * m
    return (t - m) * lax.rsqrt(v + 1e-5) * g + b


def _vnmlp_body(pool_ref, vn_ref, w1_ref, b1_ref, g1_ref, bb1_ref,
                w2_ref, b2_ref, g2_ref, bb2_ref, out_ref):
    vt = pool_ref[...] + vn_ref[...]
    t = jnp.dot(vt, w1_ref[...], preferred_element_type=jnp.float32) + b1_ref[...]
    t = jax.nn.relu(_bn_small(t, g1_ref[...], bb1_ref[...]))
    t = jnp.dot(t, w2_ref[...], preferred_element_type=jnp.float32) + b2_ref[...]
    t = jax.nn.relu(_bn_small(t, g2_ref[...], bb2_ref[...]))
    out_ref[...] = vn_ref[...] + t


_hspec = pl.BlockSpec((2, RB, H), lambda i: (0, i, 0))
_wspec = pl.BlockSpec((D, D), lambda i: (0, 0))
_bspec = pl.BlockSpec((1, D), lambda i: (0, 0))
_stspec = pl.BlockSpec((2, D), lambda i: (0, 0))
_batchspec = pl.BlockSpec((1, RB), lambda i: (0, i))
_h_sds = jax.ShapeDtypeStruct((2, NPAD, H), jnp.float32)
_st_sds = jax.ShapeDtypeStruct((2, D), jnp.float32)
_g_sds = jax.ShapeDtypeStruct((G, D), jnp.float32)
_gspec = pl.BlockSpec((G, D), lambda i: (0, 0))


def _c1(eps, hin, aggr, w, b):
    return pl.pallas_call(
        _c1_body,
        grid=(NBLK,),
        in_specs=[pl.BlockSpec((1, 1), lambda i: (0, 0)),
                  _hspec, _hspec, _wspec, _bspec],
        out_specs=[_hspec, _stspec],
        out_shape=[_h_sds, _st_sds],
    )(eps, hin, aggr, w, b)


def _c2(t1, st, g, bb, w, b):
    return pl.pallas_call(
        _c2_body,
        grid=(NBLK,),
        in_specs=[_hspec, _stspec, _bspec, _bspec, _wspec, _bspec],
        out_specs=[_hspec, _stspec],
        out_shape=[_h_sds, _st_sds],
    )(t1, st, g, bb, w, b)


def _c3(t2, st, g, bb, hin, vnn, batchp, last):
    if last:
        return pl.pallas_call(
            functools.partial(
                lambda a, b_, c_, d_, e_, f_, o: _c3_body(
                    a, b_, c_, d_, e_, None, f_, o, last=True)),
            grid=(NBLK,),
            in_specs=[_hspec, _stspec, _bspec, _bspec, _hspec, _batchspec],
            out_specs=_hspec,
            out_shape=_h_sds,
        )(t2, st, g, bb, hin, batchp)
    return pl.pallas_call(
        functools.partial(_c3_body, last=False),
        grid=(NBLK,),
        in_specs=[_hspec, _stspec, _bspec, _bspec, _hspec, _gspec, _batchspec],
        out_specs=_hspec,
        out_shape=_h_sds,
    )(t2, st, g, bb, hin, vnn, batchp)


def _pool(hin, batchp):
    return pl.pallas_call(
        _pool_body,
        grid=(NBLK,),
        in_specs=[_hspec, _batchspec],
        out_specs=_gspec,
        out_shape=_g_sds,
    )(hin, batchp)


def _vnmlp(pool, vn, w1, b1, g1, bb1, w2, b2, g2, bb2):
    one = pl.BlockSpec((G, D), lambda: (0, 0))
    bs = pl.BlockSpec((1, D), lambda: (0, 0))
    ws = pl.BlockSpec((D, D), lambda: (0, 0))
    return pl.pallas_call(
        _vnmlp_body,
        in_specs=[one, one, ws, bs, bs, bs, ws, bs, bs, bs],
        out_specs=one,
        out_shape=_g_sds,
    )(pool, vn, w1, b1, g1, bb1, w2, b2, g2, bb2)


def kernel(x, edge_index, edge_attr, batch, atom_emb, vn_emb, conv_eps,
           bond_emb, conv_W1, conv_b1, conv_bn_g, conv_bn_b, conv_W2, conv_b2,
           outer_bn_g, outer_bn_b, vn_W1, vn_b1, vn_bn1_g, vn_bn1_b,
           vn_W2, vn_b2, vn_bn2_g, vn_bn2_b):
    f32 = jnp.float32
    i32 = jnp.int32

    xT = jnp.pad(x.astype(i32).T, ((0, 0), (0, NPAD - N)))
    xg = xT.reshape(NA, NT, AC, CHA).transpose(1, 0, 2, 3)
    src = edge_index[0].astype(i32)
    dst = edge_index[1].astype(i32)
    code = (edge_attr[:, 0] * (VB * VB) + edge_attr[:, 1] * VB
            + edge_attr[:, 2]).astype(i32)
    pad = ((0, 0), (0, EPTP - EPT))
    srcp = jnp.pad(src.reshape(NT, EPT), pad)
    dstp = jnp.pad(dst.reshape(NT, EPT), pad, constant_values=NPAD - 1)
    codep = jnp.pad(code.reshape(NT, EPT), pad)
    ega = jnp.stack([srcp, codep], 1).reshape(NT, 2, NGRP, GRP * CH)
    ega = ega.transpose(0, 2, 1, 3)
    egb = dstp.reshape(NT, NGRP, GRP, CH)
    batchp = jnp.pad(batch.astype(i32), (0, NPAD - N),
                     constant_values=G).reshape(1, NPAD)
    zrows = jnp.zeros((RPT, H), f32)

    atab = atom_emb.at[0].add(vn_emb[0][None, :])
    atabs = jnp.stack([atab[:, :, :H], atab[:, :, H:]])
    et = (bond_emb[:, 0][:, :, None, None, :]
          + bond_emb[:, 1][:, None, :, None, :]
          + bond_emb[:, 2][:, None, None, :, :]).reshape(L, VB ** NB, D)
    ets = jnp.stack([et[..., :H], et[..., H:]], axis=1)

    vn = jnp.broadcast_to(vn_emb[0], (G, D)).astype(f32)
    b1r = conv_b1.reshape(L, 1, D)
    b2r = conv_b2.reshape(L, 1, D)

    hin = _atom_sc(xg, atabs)
    hout = None
    for l in range(L):
        vn_next = None
        if l < L - 1:
            pool = _pool(hin, batchp)
            vn_next = _vnmlp(pool, vn,
                             vn_W1[l], vn_b1[l].reshape(1, D),
                             vn_bn1_g[l].reshape(1, D), vn_bn1_b[l].reshape(1, D),
                             vn_W2[l], vn_b2[l].reshape(1, D),
                             vn_bn2_g[l].reshape(1, D), vn_bn2_b[l].reshape(1, D))
        aggr = _edge_sc(hin, ega, egb, ets[l], zrows)
        eps = (1.0 + conv_eps[l]).reshape(1, 1)
        t1, st1 = _c1(eps, hin, aggr, conv_W1[l], b1r[l])
        t2, st2 = _c2(t1, st1, conv_bn_g[l].reshape(1, D),
                      conv_bn_b[l].reshape(1, D), conv_W2[l], b2r[l])
        if l < L - 1:
            hin = _c3(t2, st2, outer_bn_g[l].reshape(1, D),
                      outer_bn_b[l].reshape(1, D), hin, vn_next, batchp,
                      last=False)
            vn = vn_next
        else:
            hout = _c3(t2, st2, outer_bn_g[l].reshape(1, D),
                       outer_bn_b[l].reshape(1, D), hin, None, batchp,
                       last=True)
    return jnp.concatenate([hout[0][:N], hout[1][:N]], axis=1)


# speedup vs baseline: 1.4281x; 1.0248x over previous
import functools

import jax
import jax.numpy as jnp
from jax import lax
from jax.experimental import pallas as pl
from jax.experimental.pallas import tpu as pltpu
from jax.experimental.pallas import tpu_sc as plsc

N = 10000
E = 160000
D = 256
L = 3
G = 64
NA, VA, NB, VB = 9, 119, 3, 6

H = 128
NT = 16
NPAD = 10240
RPT = NPAD // NT
EPT = E // NT
EPTP = 10240
CH = 128
NCH = EPTP // CH
GRP = 16
NGRP = NCH // GRP
AC = 8
CHA = RPT // AC

_mesh = plsc.VectorSubcoreMesh(core_axis_name="c", subcore_axis_name="s")


@functools.partial(
    pl.kernel,
    out_type=jax.ShapeDtypeStruct((2, NPAD, H), jnp.float32),
    mesh=_mesh,
    scratch_types=[
        pltpu.VMEM((NA, AC, CHA), jnp.int32),
        pltpu.VMEM((RPT, H), jnp.float32),
        pltpu.SemaphoreType.DMA,
        pltpu.SemaphoreType.DMA,
    ],
)
def _atom_sc(xg_hbm, tab_hbm, out_hbm, idx_v, buf, sem_t, sem_a):
    c = lax.axis_index("c")
    s = lax.axis_index("s")
    pltpu.sync_copy(xg_hbm.at[s], idx_v)

    def _f0(k):
        pltpu.async_copy(
            tab_hbm.at[c, 0].at[idx_v.at[0, k]],
            buf.at[pl.ds(k * CHA, CHA)], sem_t)

    _f0(0)

    def chunk(k, carry):
        pltpu.make_async_copy(
            tab_hbm.at[c, 0].at[idx_v.at[0, k]],
            buf.at[pl.ds(k * CHA, CHA)], sem_t).wait()

        @pl.when(k + 1 < AC)
        def _():
            _f0(k + 1)

        for f in range(1, NA):
            pltpu.async_copy(
                tab_hbm.at[c, f].at[idx_v.at[f, k]],
                buf.at[pl.ds(k * CHA, CHA)], sem_a, add=True)
        return carry

    lax.fori_loop(0, AC, chunk, 0)

    def drain(k, carry):
        pltpu.make_async_copy(
            tab_hbm.at[c, 1].at[idx_v.at[1, 0]],
            buf.at[pl.ds(0, CHA)], sem_a).wait()
        return carry

    lax.fori_loop(0, AC * (NA - 1), drain, 0)
    pltpu.sync_copy(buf, out_hbm.at[c, pl.ds(s * RPT, RPT)])


@functools.partial(
    pl.kernel,
    out_type=jax.ShapeDtypeStruct((2, NPAD, H), jnp.float32),
    mesh=_mesh,
    scratch_types=[
        pltpu.VMEM((2, GRP * CH), jnp.int32),
        pltpu.VMEM((GRP, CH), jnp.int32),
        pltpu.VMEM_SHARED((216, H), jnp.float32),
        pltpu.VMEM((2, CH, H), jnp.float32),
        pltpu.VMEM_SHARED((NPAD, H), jnp.float32),
        pltpu.SemaphoreType.DMA,
        pltpu.SemaphoreType.DMA,
    ],
)
def _edge_sc(hin_hbm, ega_hbm, egb_hbm, tab_hbm, zrows_hbm, out_hbm,
             idxa, idxd, tabv, buf, aggr, sem_t, sem_a):
    c = lax.axis_index("c")
    s = lax.axis_index("s")
    @pl.when(s == 0)
    def _():
        pltpu.sync_copy(tab_hbm.at[c], tabv)

    pltpu.sync_copy(zrows_hbm, aggr.at[pl.ds(s * RPT, RPT)])
    plsc.subcore_barrier()

    def _relu(p):
        def rrow(r, cc):
            for j in range(H // 16):
                sl = (p, r, pl.ds(j * 16, 16))
                buf[sl] = jnp.maximum(buf[sl], 0.0)
            return cc

        lax.fori_loop(0, CH, rrow, 0)

    def _tab(u):
        return pltpu.async_copy(
            tabv.at[idxa.at[1, pl.ds(u * CH, CH)]], buf.at[u % 2], sem_t)

    def _gather(u):
        return pltpu.async_copy(
            hin_hbm.at[c].at[idxa.at[0, pl.ds(u * CH, CH)]],
            buf.at[u % 2], sem_a, add=True)

    def body(i, carry):
        pltpu.sync_copy(ega_hbm.at[s, i], idxa)
        pltpu.sync_copy(egb_hbm.at[s, i], idxd)
        dt = [_tab(0), _tab(1)]
        da = []
        dt[0].wait()
        da.append(_gather(0))
        dt[1].wait()
        da.append(_gather(1))
        for u in range(GRP):
            da[u].wait()
            _relu(u % 2)
            pltpu.sync_copy(buf.at[u % 2], aggr.at[idxd.at[u]], add=True)
            if u + 2 < GRP:
                dt.append(_tab(u + 2))
                dt[u + 2].wait()
                da.append(_gather(u + 2))
        return carry

    lax.fori_loop(0, NGRP, body, 0)
    plsc.subcore_barrier()
    pltpu.sync_copy(aggr.at[pl.ds(s * RPT, RPT)],
                    out_hbm.at[c, pl.ds(s * RPT, RPT)])


RB = 512
NBLK = NPAD // RB


def _cat(ref):
    b = ref[...]
    return jnp.concatenate([b[0], b[1]], axis=-1)


def _rowmask(i):
    rows = i * RB + lax.broadcasted_iota(jnp.int32, (RB, 1), 0)
    return rows < N


def _store_halves(ref, v):
    ref[0] = v[:, :H]
    ref[1] = v[:, H:]


def _masked_stats(i, st_ref, t):
    tm = jnp.where(_rowmask(i), t, 0.0)
    st = jnp.stack([jnp.sum(tm, axis=0), jnp.sum(tm * tm, axis=0)])

    @pl.when(i == 0)
    def _():
        st_ref[...] = st

    @pl.when(i > 0)
    def _():
        st_ref[...] += st


def _c1_body(eps_ref, hin_ref, aggr_ref, w_ref, b_ref, t1_ref, st_ref):
    i = pl.program_id(0)
    z = eps_ref[0, 0] * _cat(hin_ref) + _cat(aggr_ref)
    t1 = jnp.dot(z, w_ref[...], preferred_element_type=jnp.float32) + b_ref[...]
    _store_halves(t1_ref, t1)
    _masked_stats(i, st_ref, t1)


def _bn_from_stats(st_ref, t, g_ref, b_ref):
    mean = st_ref[0:1, :] / N
    var = st_ref[1:2, :] / N - mean * mean
    inv = lax.rsqrt(var + 1e-5)
    return (t - mean) * inv * g_ref[...] + b_ref[...]


def _c2_body(t1_ref, st_ref, g_ref, bb_ref, w_ref, b_ref, t2_ref, st2_ref):
    i = pl.program_id(0)
    y = jax.nn.relu(_bn_from_stats(st_ref, _cat(t1_ref), g_ref, bb_ref))
    t2 = jnp.dot(y, w_ref[...], preferred_element_type=jnp.float32) + b_ref[...]
    _store_halves(t2_ref, t2)
    _masked_stats(i, st2_ref, t2)


def _c3_body(t2_ref, st_ref, g_ref, bb_ref, hin_ref, vnn_ref, batch_ref,
             out_ref, *, last):
    h = _bn_from_stats(st_ref, _cat(t2_ref), g_ref, bb_ref)
    if not last:
        h = jax.nn.relu(h)
    h = h + _cat(hin_ref)
    if not last:
        onehot = (batch_ref[0][:, None]
                  == lax.broadcasted_iota(jnp.int32, (1, G), 1)
                  ).astype(jnp.float32)
        h = h + jnp.dot(onehot, vnn_ref[...],
                        preferred_element_type=jnp.float32)
    _store_halves(out_ref, h)


def _pool_body(hin_ref, batch_ref, out_ref):
    i = pl.program_id(0)
    onehot_t = (lax.broadcasted_iota(jnp.int32, (G, 1), 0)
                == batch_ref[0][None, :]).astype(jnp.float32)
    p = jnp.dot(onehot_t, _cat(hin_ref), preferred_element_type=jnp.float32)

    @pl.when(i == 0)
    def _():
        out_ref[...] = p

    @pl.when(i > 0)
    def _():
        out_ref[...] += p


def _bn_small(t, g, b):
    m = jnp.mean(t, axis=0, keepdims=True)
    v = jnp.mean(t * t, axis=0, keepdims=True) - m * m
    return (t - m) * lax.rsqrt(v + 1e-5) * g + b


def _vnmlp_body(pool_ref, vn_ref, w1_ref, b1_ref, g1_ref, bb1_ref,
                w2_ref, b2_ref, g2_ref, bb2_ref, out_ref):
    vt = pool_ref[...] + vn_ref[...]
    t = jnp.dot(vt, w1_ref[...], preferred_element_type=jnp.float32) + b1_ref[...]
    t = jax.nn.relu(_bn_small(t, g1_ref[...], bb1_ref[...]))
    t = jnp.dot(t, w2_ref[...], preferred_element_type=jnp.float32) + b2_ref[...]
    t = jax.nn.relu(_bn_small(t, g2_ref[...], bb2_ref[...]))
    out_ref[...] = vn_ref[...] + t


_hspec = pl.BlockSpec((2, RB, H), lambda i: (0, i, 0))
_wspec = pl.BlockSpec((D, D), lambda i: (0, 0))
_bspec = pl.BlockSpec((1, D), lambda i: (0, 0))
_stspec = pl.BlockSpec((2, D), lambda i: (0, 0))
_batchspec = pl.BlockSpec((1, RB), lambda i: (0, i))
_h_sds = jax.ShapeDtypeStruct((2, NPAD, H), jnp.float32)
_st_sds = jax.ShapeDtypeStruct((2, D), jnp.float32)
_g_sds = jax.ShapeDtypeStruct((G, D), jnp.float32)
_gspec = pl.BlockSpec((G, D), lambda i: (0, 0))


def _c1(eps, hin, aggr, w, b):
    return pl.pallas_call(
        _c1_body,
        grid=(NBLK,),
        in_specs=[pl.BlockSpec((1, 1), lambda i: (0, 0)),
                  _hspec, _hspec, _wspec, _bspec],
        out_specs=[_hspec, _stspec],
        out_shape=[_h_sds, _st_sds],
    )(eps, hin, aggr, w, b)


def _c2(t1, st, g, bb, w, b):
    return pl.pallas_call(
        _c2_body,
        grid=(NBLK,),
        in_specs=[_hspec, _stspec, _bspec, _bspec, _wspec, _bspec],
        out_specs=[_hspec, _stspec],
        out_shape=[_h_sds, _st_sds],
    )(t1, st, g, bb, w, b)


def _c3(t2, st, g, bb, hin, vnn, batchp, last):
    if last:
        return pl.pallas_call(
            functools.partial(
                lambda a, b_, c_, d_, e_, f_, o: _c3_body(
                    a, b_, c_, d_, e_, None, f_, o, last=True)),
            grid=(NBLK,),
            in_specs=[_hspec, _stspec, _bspec, _bspec, _hspec, _batchspec],
            out_specs=_hspec,
            out_shape=_h_sds,
        )(t2, st, g, bb, hin, batchp)
    return pl.pallas_call(
        functools.partial(_c3_body, last=False),
        grid=(NBLK,),
        in_specs=[_hspec, _stspec, _bspec, _bspec, _hspec, _gspec, _batchspec],
        out_specs=_hspec,
        out_shape=_h_sds,
    )(t2, st, g, bb, hin, vnn, batchp)


def _pool(hin, batchp):
    return pl.pallas_call(
        _pool_body,
        grid=(NBLK,),
        in_specs=[_hspec, _batchspec],
        out_specs=_gspec,
        out_shape=_g_sds,
    )(hin, batchp)


def _vnmlp(pool, vn, w1, b1, g1, bb1, w2, b2, g2, bb2):
    one = pl.BlockSpec((G, D), lambda: (0, 0))
    bs = pl.BlockSpec((1, D), lambda: (0, 0))
    ws = pl.BlockSpec((D, D), lambda: (0, 0))
    return pl.pallas_call(
        _vnmlp_body,
        in_specs=[one, one, ws, bs, bs, bs, ws, bs, bs, bs],
        out_specs=one,
        out_shape=_g_sds,
    )(pool, vn, w1, b1, g1, bb1, w2, b2, g2, bb2)


def kernel(x, edge_index, edge_attr, batch, atom_emb, vn_emb, conv_eps,
           bond_emb, conv_W1, conv_b1, conv_bn_g, conv_bn_b, conv_W2, conv_b2,
           outer_bn_g, outer_bn_b, vn_W1, vn_b1, vn_bn1_g, vn_bn1_b,
           vn_W2, vn_b2, vn_bn2_g, vn_bn2_b):
    f32 = jnp.float32
    i32 = jnp.int32

    xT = jnp.pad(x.astype(i32).T, ((0, 0), (0, NPAD - N)))
    xg = xT.reshape(NA, NT, AC, CHA).transpose(1, 0, 2, 3)
    src = edge_index[0].astype(i32)
    dst = edge_index[1].astype(i32)
    code = (edge_attr[:, 0] * (VB * VB) + edge_attr[:, 1] * VB
            + edge_attr[:, 2]).astype(i32)
    pad = ((0, 0), (0, EPTP - EPT))
    srcp = jnp.pad(src.reshape(NT, EPT), pad)
    dstp = jnp.pad(dst.reshape(NT, EPT), pad, constant_values=NPAD - 1)
    codep = jnp.pad(code.reshape(NT, EPT), pad)
    ega = jnp.stack([srcp, codep], 1).reshape(NT, 2, NGRP, GRP * CH)
    ega = ega.transpose(0, 2, 1, 3)
    egb = dstp.reshape(NT, NGRP, GRP, CH)
    batchp = jnp.pad(batch.astype(i32), (0, NPAD - N),
                     constant_values=G).reshape(1, NPAD)
    zrows = jnp.zeros((RPT, H), f32)

    atab = atom_emb.at[0].add(vn_emb[0][None, :])
    atabs = jnp.stack([atab[:, :, :H], atab[:, :, H:]])
    et = (bond_emb[:, 0][:, :, None, None, :]
          + bond_emb[:, 1][:, None, :, None, :]
          + bond_emb[:, 2][:, None, None, :, :]).reshape(L, VB ** NB, D)
    ets = jnp.stack([et[..., :H], et[..., H:]], axis=1)

    vn = jnp.broadcast_to(vn_emb[0], (G, D)).astype(f32)
    b1r = conv_b1.reshape(L, 1, D)
    b2r = conv_b2.reshape(L, 1, D)

    hin = _atom_sc(xg, atabs)
    hout = None
    for l in range(L):
        vn_next = None
        if l < L - 1:
            pool = _pool(hin, batchp)
            vn_next = _vnmlp(pool, vn,
                             vn_W1[l], vn_b1[l].reshape(1, D),
                             vn_bn1_g[l].reshape(1, D), vn_bn1_b[l].reshape(1, D),
                             vn_W2[l], vn_b2[l].reshape(1, D),
                             vn_bn2_g[l].reshape(1, D), vn_bn2_b[l].reshape(1, D))
        aggr = _edge_sc(hin, ega, egb, ets[l], zrows)
        eps = (1.0 + conv_eps[l]).reshape(1, 1)
        t1, st1 = _c1(eps, hin, aggr, conv_W1[l], b1r[l])
        t2, st2 = _c2(t1, st1, conv_bn_g[l].reshape(1, D),
                      conv_bn_b[l].reshape(1, D), conv_W2[l], b2r[l])
        if l < L - 1:
            hin = _c3(t2, st2, outer_bn_g[l].reshape(1, D),
                      outer_bn_b[l].reshape(1, D), hin, vn_next, batchp,
                      last=False)
            vn = vn_next
        else:
            hout = _c3(t2, st2, outer_bn_g[l].reshape(1, D),
                       outer_bn_b[l].reshape(1, D), hin, None, batchp,
                       last=True)
    return jnp.concatenate([hout[0][:N], hout[1][:N]], axis=1)
